# Initial kernel scaffold; baseline (speedup 1.0000x reference)
import functools

import jax
import jax.numpy as jnp
from jax import lax
from jax.experimental import pallas as pl
from jax.experimental.pallas import tpu as pltpu
from jax.experimental.pallas import tpu_sc as plsc

_N = 10000
_E = 320000
_IN = 128
_H = 8
_OUT = 16
_D = _H * _OUT
_HID = 128
_CHUNK = 128
_R = _E // _CHUNK
_NC = 2
_NS = 16
_NW = _NC * _NS
_ROWS_PER_SUB = _N // _NS
_LEAK = 0.2
_BLK = 2000
_GRID = _N // _BLK


def _pre_body(x_ref, w0_ref, w1_ref, ml0_ref, mr0_ref, ml1_ref, mr1_ref,
              feat0_ref, feat1_ref, tl0_ref, tr0_ref, tl1_ref, tr1_ref, c_ref,
              acc_ref):
    i = pl.program_id(0)
    x = x_ref[...]
    f0 = jnp.dot(x, w0_ref[...], preferred_element_type=jnp.float32)
    f1 = jnp.dot(x, w1_ref[...], preferred_element_type=jnp.float32)
    feat0_ref[...] = f0
    feat1_ref[...] = f1
    tl0 = jnp.dot(f0, ml0_ref[...], preferred_element_type=jnp.float32)
    tr0 = jnp.dot(f0, mr0_ref[...], preferred_element_type=jnp.float32)
    tl1 = jnp.dot(f1, ml1_ref[...], preferred_element_type=jnp.float32)
    tr1 = jnp.dot(f1, mr1_ref[...], preferred_element_type=jnp.float32)
    tl0_ref[...] = tl0
    tr0_ref[...] = tr0
    tl1_ref[...] = tl1
    tr1_ref[...] = tr1
    for row, t in enumerate((tl0, tr0, tl1, tr1)):
        m = jnp.max(t, axis=0)
        prev = acc_ref[row, :]
        acc_ref[row, :] = jnp.where(i == 0, m, jnp.maximum(prev, m))
    zero = jnp.zeros((16,), jnp.float32)
    c_ref[0, :] = jnp.maximum(zero, acc_ref[0, :] + acc_ref[1, :])
    c_ref[1, :] = jnp.maximum(zero, acc_ref[2, :] + acc_ref[3, :])


def _pre(x, w0, w1, ml0, mr0, ml1, mr1):
    blk = _BLK
    return pl.pallas_call(
        _pre_body,
        grid=(_GRID,),
        in_specs=[
            pl.BlockSpec((blk, _IN), lambda i: (i, 0)),
            pl.BlockSpec((_IN, _D), lambda i: (0, 0)),
            pl.BlockSpec((_IN, _D), lambda i: (0, 0)),
            pl.BlockSpec((_D, 16), lambda i: (0, 0)),
            pl.BlockSpec((_D, 16), lambda i: (0, 0)),
            pl.BlockSpec((_D, 16), lambda i: (0, 0)),
            pl.BlockSpec((_D, 16), lambda i: (0, 0)),
        ],
        out_specs=[
            pl.BlockSpec((blk, _D), lambda i: (i, 0)),
            pl.BlockSpec((blk, _D), lambda i: (i, 0)),
            pl.BlockSpec((blk, 16), lambda i: (i, 0)),
            pl.BlockSpec((blk, 16), lambda i: (i, 0)),
            pl.BlockSpec((blk, 16), lambda i: (i, 0)),
            pl.BlockSpec((blk, 16), lambda i: (i, 0)),
            pl.BlockSpec((2, 16), lambda i: (0, 0)),
        ],
        out_shape=[
            jax.ShapeDtypeStruct((_N, _D), jnp.float32),
            jax.ShapeDtypeStruct((_N, _D), jnp.float32),
            jax.ShapeDtypeStruct((_N, 16), jnp.float32),
            jax.ShapeDtypeStruct((_N, 16), jnp.float32),
            jax.ShapeDtypeStruct((_N, 16), jnp.float32),
            jax.ShapeDtypeStruct((_N, 16), jnp.float32),
            jax.ShapeDtypeStruct((2, 16), jnp.float32),
        ],
        scratch_shapes=[pltpu.VMEM((4, 16), jnp.float32)],
    )(x, w0, w1, ml0, mr0, ml1, mr1)


def _p1_body(src_ref, dst_ref, tl_ref, tr_ref, c_ref,
             ee_ref, dp_ref,
             sidx, didx, abuf, bbuf, eebuf, cbuf, dsh, sem1, sem2):
    cid = lax.axis_index("c")
    sid = lax.axis_index("s")
    wid = sid * _NC + cid

    def _z(i, _):
        eebuf[i, :] = jnp.zeros((16,), jnp.float32)
        return 0
    lax.fori_loop(0, _CHUNK, _z, 0)
    for k in range(5):
        pltpu.sync_copy(eebuf.at[pl.ds(0, 125)],
                        dsh.at[pl.ds(sid * _ROWS_PER_SUB + k * 125, 125)])
    plsc.subcore_barrier()

    pltpu.sync_copy(c_ref, cbuf)
    cvec = cbuf[0, :]

    nb = jnp.where(wid < _R - (_R // _NW) * _NW, _R // _NW + 1, _R // _NW)

    def _chunk(i, _):
        r = wid + _NW * i
        pltpu.sync_copy(src_ref.at[r], sidx)
        pltpu.sync_copy(dst_ref.at[r], didx)
        cp1 = pltpu.async_copy(tl_ref.at[sidx], abuf, sem1)
        cp2 = pltpu.async_copy(tr_ref.at[didx], bbuf, sem2)
        cp1.wait()
        cp2.wait()

        def _edge(j, _):
            e = abuf[j, :] + bbuf[j, :]
            e = jnp.where(e > 0.0, e, _LEAK * e) - cvec
            eebuf[j, :] = jnp.exp(e)
            return 0
        lax.fori_loop(0, _CHUNK, _edge, 0)
        pltpu.sync_copy(eebuf, ee_ref.at[r])
        pltpu.sync_copy(eebuf, dsh.at[didx], add=True)
        return 0
    lax.fori_loop(0, nb, _chunk, 0)

    plsc.subcore_barrier()
    for k in range(5):
        off = sid * _ROWS_PER_SUB + k * 125
        pltpu.sync_copy(dsh.at[pl.ds(off, 125)], eebuf.at[pl.ds(0, 125)])
        pltpu.sync_copy(eebuf.at[pl.ds(0, 125)], dp_ref.at[cid, pl.ds(off, 125)])


def _p1(src2, dst2, tl, tr, c):
    mesh = plsc.VectorSubcoreMesh(core_axis_name="c", subcore_axis_name="s")
    f = pl.kernel(
        _p1_body,
        out_type=[
            jax.ShapeDtypeStruct((_R, _CHUNK, 16), jnp.float32),
            jax.ShapeDtypeStruct((_NC, _N, 16), jnp.float32),
        ],
        mesh=mesh,
        scratch_types=[
            pltpu.VMEM((_CHUNK,), jnp.int32),
            pltpu.VMEM((_CHUNK,), jnp.int32),
            pltpu.VMEM((_CHUNK, 16), jnp.float32),
            pltpu.VMEM((_CHUNK, 16), jnp.float32),
            pltpu.VMEM((_CHUNK, 16), jnp.float32),
            pltpu.VMEM((1, 16), jnp.float32),
            pltpu.VMEM_SHARED((_N, 16), jnp.float32),
            pltpu.SemaphoreType.DMA,
            pltpu.SemaphoreType.DMA,
        ],
    )
    return f(src2, dst2, tl, tr, c)


def _rd_body(dp_ref, rd_ref):
    rd_ref[...] = 1.0 / (dp_ref[0] + dp_ref[1] + 1e-30)


def _rd(dp):
    return pl.pallas_call(
        _rd_body,
        out_shape=jax.ShapeDtypeStruct((_N, 16), jnp.float32),
    )(dp)


def _p2_body(src_ref, dst_ref, feat_ref, ee_ref, rd_ref,
             op_ref,
             sidx, didx, featbuf, eebuf, rdbuf, osh, sem1, sem2):
    cid = lax.axis_index("c")
    sid = lax.axis_index("s")
    wid = sid * _NC + cid

    def _z(i, _):
        for h in range(_H):
            featbuf[i, pl.ds(16 * h, 16)] = jnp.zeros((16,), jnp.float32)
        return 0
    lax.fori_loop(0, _CHUNK, _z, 0)
    for k in range(5):
        pltpu.sync_copy(featbuf.at[pl.ds(0, 125)],
                        osh.at[pl.ds(sid * _ROWS_PER_SUB + k * 125, 125)])
    plsc.subcore_barrier()

    nb = jnp.where(wid < _R - (_R // _NW) * _NW, _R // _NW + 1, _R // _NW)

    def _chunk(i, _):
        r = wid + _NW * i
        pltpu.sync_copy(src_ref.at[r], sidx)
        pltpu.sync_copy(dst_ref.at[r], didx)
        cp1 = pltpu.async_copy(feat_ref.at[sidx], featbuf, sem1)
        pltpu.sync_copy(ee_ref.at[r], eebuf)
        cp2 = pltpu.async_copy(rd_ref.at[didx], rdbuf, sem2)
        cp1.wait()
        cp2.wait()

        def _edge(j, _):
            a = eebuf[j, :] * rdbuf[j, :]
            eebuf[j, :] = a
            for h in range(_H):
                s = eebuf[j, h]
                featbuf[j, pl.ds(16 * h, 16)] = featbuf[j, pl.ds(16 * h, 16)] * s
            return 0
        lax.fori_loop(0, _CHUNK, _edge, 0)
        pltpu.sync_copy(featbuf, osh.at[didx], add=True)
        return 0
    lax.fori_loop(0, nb, _chunk, 0)

    plsc.subcore_barrier()
    for k in range(5):
        off = sid * _ROWS_PER_SUB + k * 125
        pltpu.sync_copy(osh.at[pl.ds(off, 125)], featbuf.at[pl.ds(0, 125)])
        pltpu.sync_copy(featbuf.at[pl.ds(0, 125)], op_ref.at[cid, pl.ds(off, 125)])


def _p2(src2, dst2, feat, ee, rd):
    mesh = plsc.VectorSubcoreMesh(core_axis_name="c", subcore_axis_name="s")
    f = pl.kernel(
        _p2_body,
        out_type=jax.ShapeDtypeStruct((_NC, _N, _D), jnp.float32),
        mesh=mesh,
        scratch_types=[
            pltpu.VMEM((_CHUNK,), jnp.int32),
            pltpu.VMEM((_CHUNK,), jnp.int32),
            pltpu.VMEM((_CHUNK, _D), jnp.float32),
            pltpu.VMEM((_CHUNK, 16), jnp.float32),
            pltpu.VMEM((_CHUNK, 16), jnp.float32),
            pltpu.VMEM_SHARED((_N, _D), jnp.float32),
            pltpu.SemaphoreType.DMA,
            pltpu.SemaphoreType.DMA,
        ],
    )
    return f(src2, dst2, feat, ee, rd)


def _ka_body(op0_ref, op1_ref, b0_ref, b1_ref, wp1_ref, bp1_ref, wp2_ref,
             z0_ref, z1_ref, w_ref):
    z0 = op0_ref[0] + op0_ref[1] + b0_ref[...]
    z1 = op1_ref[0] + op1_ref[1] + b1_ref[...]
    z0_ref[...] = z0
    z1_ref[...] = z1
    t0 = jnp.tanh(jnp.dot(z0, wp1_ref[...], preferred_element_type=jnp.float32)
                  + bp1_ref[...])
    t1 = jnp.tanh(jnp.dot(z1, wp1_ref[...], preferred_element_type=jnp.float32)
                  + bp1_ref[...])
    w0 = jnp.sum(t0 * wp2_ref[...], axis=1, keepdims=True)
    w1 = jnp.sum(t1 * wp2_ref[...], axis=1, keepdims=True)
    w_ref[...] = jnp.concatenate([w0, w1], axis=1)


def _ka(op0, op1, b0r, b1r, wp1, bp1r, wp2r):
    blk = _BLK
    return pl.pallas_call(
        _ka_body,
        grid=(_GRID,),
        in_specs=[
            pl.BlockSpec((_NC, blk, _D), lambda i: (0, i, 0)),
            pl.BlockSpec((_NC, blk, _D), lambda i: (0, i, 0)),
            pl.BlockSpec((1, _D), lambda i: (0, 0)),
            pl.BlockSpec((1, _D), lambda i: (0, 0)),
            pl.BlockSpec((_D, _HID), lambda i: (0, 0)),
            pl.BlockSpec((1, _HID), lambda i: (0, 0)),
            pl.BlockSpec((1, _HID), lambda i: (0, 0)),
        ],
        out_specs=[
            pl.BlockSpec((blk, _D), lambda i: (i, 0)),
            pl.BlockSpec((blk, _D), lambda i: (i, 0)),
            pl.BlockSpec((blk, 2), lambda i: (i, 0)),
        ],
        out_shape=[
            jax.ShapeDtypeStruct((_N, _D), jnp.float32),
            jax.ShapeDtypeStruct((_N, _D), jnp.float32),
            jax.ShapeDtypeStruct((_N, 2), jnp.float32),
        ],
    )(op0, op1, b0r, b1r, wp1, bp1r, wp2r)


def _kb_body(z0_ref, z1_ref, w_ref, out_ref):
    w = w_ref[...]
    s0 = jnp.sum(w[:, 0:1]) / _N
    s1 = jnp.sum(w[:, 1:2]) / _N
    m = jnp.maximum(s0, s1)
    e0 = jnp.exp(s0 - m)
    e1 = jnp.exp(s1 - m)
    beta0 = e0 / (e0 + e1)
    beta1 = e1 / (e0 + e1)
    out_ref[...] = beta0 * z0_ref[...] + beta1 * z1_ref[...]


def _kb(z0, z1, w):
    blk = _BLK
    return pl.pallas_call(
        _kb_body,
        grid=(_GRID,),
        in_specs=[
            pl.BlockSpec((blk, _D), lambda i: (i, 0)),
            pl.BlockSpec((blk, _D), lambda i: (i, 0)),
            pl.BlockSpec((_N, 2), lambda i: (0, 0)),
        ],
        out_specs=pl.BlockSpec((blk, _D), lambda i: (i, 0)),
        out_shape=jax.ShapeDtypeStruct((_N, _D), jnp.float32),
    )(z0, z1, w)


def _attn_mats(attn_l, attn_r):
    k = jnp.arange(_D)
    hp = jnp.arange(16)
    mask = (k[:, None] // _OUT) == (hp[None, :] % _H)
    ml = jnp.where(mask, attn_l.reshape(_D)[:, None], 0.0)
    mr = jnp.where(mask, attn_r.reshape(_D)[:, None], 0.0)
    return ml.astype(jnp.float32), mr.astype(jnp.float32)


def kernel(x, edge_index_0, edge_index_1, W0, attn_l0, attn_r0, b0,
           W1, attn_l1, attn_r1, b1, Wp1, bp1, Wp2):
    src0 = edge_index_0[0].reshape(_R, _CHUNK)
    dst0 = edge_index_0[1].reshape(_R, _CHUNK)
    src1 = edge_index_1[0].reshape(_R, _CHUNK)
    dst1 = edge_index_1[1].reshape(_R, _CHUNK)

    ml0, mr0 = _attn_mats(attn_l0, attn_r0)
    ml1, mr1 = _attn_mats(attn_l1, attn_r1)

    feat0, feat1, tl0, tr0, tl1, tr1, c = _pre(x, W0, W1, ml0, mr0, ml1, mr1)

    ee0, dp0 = _p1(src0, dst0, tl0, tr0, c[0:1])
    ee1, dp1 = _p1(src1, dst1, tl1, tr1, c[1:2])

    rd0 = _rd(dp0)
    rd1 = _rd(dp1)

    op0 = _p2(src0, dst0, feat0, ee0, rd0)
    op1 = _p2(src1, dst1, feat1, ee1, rd1)

    z0, z1, w = _ka(op0, op1, b0.reshape(1, _D), b1.reshape(1, _D),
                    Wp1, bp1.reshape(1, _HID), Wp2.reshape(1, _HID))
    return _kb(z0, z1, w)


# speedup vs baseline: 67.2313x; 67.2313x over previous
import jax
import jax.numpy as jnp
from jax import lax
from jax.experimental import pallas as pl
from jax.experimental.pallas import tpu as pltpu
from jax.experimental.pallas import tpu_sc as plsc

_N = 10000
_E = 320000
_IN = 128
_H = 8
_OUT = 16
_D = _H * _OUT
_HID = 128
_CHUNK = 128
_R = _E // _CHUNK
_NC = 2
_NS = 16
_NW = _NC * _NS
_SUB_BASE = 624
_SUB_CHUNKS = ((0, 128), (128, 128), (256, 128), (384, 128), (512, 112))
_TAIL_OFF = _SUB_BASE * _NS
_TAIL = _N - _TAIL_OFF
_LEAK = 0.2
_BLK = 2000
_GRID = _N // _BLK


def _pre_body(x_ref, w0_ref, w1_ref, ml0_ref, mr0_ref, ml1_ref, mr1_ref,
              feat0_ref, feat1_ref, tl0_ref, tr0_ref, tl1_ref, tr1_ref, c_ref,
              acc_ref):
    i = pl.program_id(0)
    x = x_ref[...]
    f0 = jnp.dot(x, w0_ref[...], preferred_element_type=jnp.float32)
    f1 = jnp.dot(x, w1_ref[...], preferred_element_type=jnp.float32)
    feat0_ref[...] = f0
    feat1_ref[...] = f1
    tl0 = jnp.dot(f0, ml0_ref[...], preferred_element_type=jnp.float32)
    tr0 = jnp.dot(f0, mr0_ref[...], preferred_element_type=jnp.float32)
    tl1 = jnp.dot(f1, ml1_ref[...], preferred_element_type=jnp.float32)
    tr1 = jnp.dot(f1, mr1_ref[...], preferred_element_type=jnp.float32)
    tl0_ref[...] = tl0
    tr0_ref[...] = tr0
    tl1_ref[...] = tl1
    tr1_ref[...] = tr1
    for row, t in enumerate((tl0, tr0, tl1, tr1)):
        m = jnp.max(t, axis=0)
        prev = acc_ref[row, :]
        acc_ref[row, :] = jnp.where(i == 0, m, jnp.maximum(prev, m))
    zero = jnp.zeros((16,), jnp.float32)
    c_ref[0, :] = jnp.maximum(zero, acc_ref[0, :] + acc_ref[1, :])
    c_ref[1, :] = jnp.maximum(zero, acc_ref[2, :] + acc_ref[3, :])


def _pre(x, w0, w1, ml0, mr0, ml1, mr1):
    blk = _BLK
    return pl.pallas_call(
        _pre_body,
        grid=(_GRID,),
        in_specs=[
            pl.BlockSpec((blk, _IN), lambda i: (i, 0)),
            pl.BlockSpec((_IN, _D), lambda i: (0, 0)),
            pl.BlockSpec((_IN, _D), lambda i: (0, 0)),
            pl.BlockSpec((_D, 16), lambda i: (0, 0)),
            pl.BlockSpec((_D, 16), lambda i: (0, 0)),
            pl.BlockSpec((_D, 16), lambda i: (0, 0)),
            pl.BlockSpec((_D, 16), lambda i: (0, 0)),
        ],
        out_specs=[
            pl.BlockSpec((blk, _D), lambda i: (i, 0)),
            pl.BlockSpec((blk, _D), lambda i: (i, 0)),
            pl.BlockSpec((blk, 16), lambda i: (i, 0)),
            pl.BlockSpec((blk, 16), lambda i: (i, 0)),
            pl.BlockSpec((blk, 16), lambda i: (i, 0)),
            pl.BlockSpec((blk, 16), lambda i: (i, 0)),
            pl.BlockSpec((2, 16), lambda i: (0, 0)),
        ],
        out_shape=[
            jax.ShapeDtypeStruct((_N, _D), jnp.float32),
            jax.ShapeDtypeStruct((_N, _D), jnp.float32),
            jax.ShapeDtypeStruct((_N, 16), jnp.float32),
            jax.ShapeDtypeStruct((_N, 16), jnp.float32),
            jax.ShapeDtypeStruct((_N, 16), jnp.float32),
            jax.ShapeDtypeStruct((_N, 16), jnp.float32),
            jax.ShapeDtypeStruct((2, 16), jnp.float32),
        ],
        scratch_shapes=[pltpu.VMEM((4, 16), jnp.float32)],
    )(x, w0, w1, ml0, mr0, ml1, mr1)


def _p1_body(src_ref, dst_ref, tl_ref, tr_ref, c_ref,
             ee_ref, dp_ref,
             sidx, didx, abuf, bbuf, eebuf, cbuf, dsh, sem1, sem2):
    cid = lax.axis_index("c")
    sid = lax.axis_index("s")
    wid = sid * _NC + cid

    def _z(i, _):
        eebuf[i, :] = jnp.zeros((16,), jnp.float32)
        return 0
    lax.fori_loop(0, _CHUNK, _z, 0)
    base = pl.multiple_of(sid * _SUB_BASE, 8)
    for off, sz in _SUB_CHUNKS:
        o = pl.multiple_of(base + off, 8)
        pltpu.sync_copy(eebuf.at[pl.ds(0, sz)], dsh.at[pl.ds(o, sz)])

    @pl.when(sid == _NS - 1)
    def _zt():
        pltpu.sync_copy(eebuf.at[pl.ds(0, _TAIL)],
                        dsh.at[pl.ds(_TAIL_OFF, _TAIL)])
    plsc.subcore_barrier()

    pltpu.sync_copy(c_ref, cbuf)
    cvec = cbuf[0, :]

    nb = jnp.where(wid < _R - (_R // _NW) * _NW, _R // _NW + 1, _R // _NW)

    def _chunk(i, _):
        r = wid + _NW * i
        pltpu.sync_copy(src_ref.at[r], sidx)
        pltpu.sync_copy(dst_ref.at[r], didx)
        cp1 = pltpu.async_copy(tl_ref.at[sidx], abuf, sem1)
        cp2 = pltpu.async_copy(tr_ref.at[didx], bbuf, sem2)
        cp1.wait()
        cp2.wait()

        def _edge(j, _):
            e = abuf[j, :] + bbuf[j, :]
            e = jnp.where(e > 0.0, e, _LEAK * e) - cvec
            eebuf[j, :] = jnp.exp(e)
            return 0
        lax.fori_loop(0, _CHUNK, _edge, 0)
        pltpu.sync_copy(eebuf, ee_ref.at[r])
        pltpu.sync_copy(eebuf, dsh.at[didx], add=True)
        return 0
    lax.fori_loop(0, nb, _chunk, 0)

    plsc.subcore_barrier()
    for off, sz in _SUB_CHUNKS:
        o = pl.multiple_of(base + off, 8)
        pltpu.sync_copy(dsh.at[pl.ds(o, sz)], dp_ref.at[cid, pl.ds(o, sz)])

    @pl.when(sid == _NS - 1)
    def _xt():
        pltpu.sync_copy(dsh.at[pl.ds(_TAIL_OFF, _TAIL)],
                        dp_ref.at[cid, pl.ds(_TAIL_OFF, _TAIL)])


def _p1(src2, dst2, tl, tr, c):
    mesh = plsc.VectorSubcoreMesh(core_axis_name="c", subcore_axis_name="s")
    f = pl.kernel(
        _p1_body,
        out_type=[
            jax.ShapeDtypeStruct((_R, _CHUNK, 16), jnp.float32),
            jax.ShapeDtypeStruct((_NC, _N, 16), jnp.float32),
        ],
        mesh=mesh,
        compiler_params=pltpu.CompilerParams(use_tc_tiling_on_sc=False),
        scratch_types=[
            pltpu.VMEM((_CHUNK,), jnp.int32),
            pltpu.VMEM((_CHUNK,), jnp.int32),
            pltpu.VMEM((_CHUNK, 16), jnp.float32),
            pltpu.VMEM((_CHUNK, 16), jnp.float32),
            pltpu.VMEM((_CHUNK, 16), jnp.float32),
            pltpu.VMEM((1, 16), jnp.float32),
            pltpu.VMEM_SHARED((_N, 16), jnp.float32),
            pltpu.SemaphoreType.DMA,
            pltpu.SemaphoreType.DMA,
        ],
    )
    return f(src2, dst2, tl, tr, c)


def _p2_body(src_ref, dst_ref, feat_ref, ee_ref,
             op_ref,
             sidx, didx, featbuf, eebuf, osh, sem1):
    cid = lax.axis_index("c")
    sid = lax.axis_index("s")
    wid = sid * _NC + cid

    def _z(i, _):
        for h in range(_H):
            featbuf[i, pl.ds(16 * h, 16)] = jnp.zeros((16,), jnp.float32)
        return 0
    lax.fori_loop(0, _CHUNK, _z, 0)
    base = pl.multiple_of(sid * _SUB_BASE, 8)
    for off, sz in _SUB_CHUNKS:
        pltpu.sync_copy(featbuf.at[pl.ds(0, sz)],
                        osh.at[pl.ds(pl.multiple_of(base + off, 8), sz)])

    @pl.when(sid == _NS - 1)
    def _zt():
        pltpu.sync_copy(featbuf.at[pl.ds(0, _TAIL)],
                        osh.at[pl.ds(_TAIL_OFF, _TAIL)])
    plsc.subcore_barrier()

    nb = jnp.where(wid < _R - (_R // _NW) * _NW, _R // _NW + 1, _R // _NW)

    def _chunk(i, _):
        r = wid + _NW * i
        pltpu.sync_copy(src_ref.at[r], sidx)
        pltpu.sync_copy(dst_ref.at[r], didx)
        cp1 = pltpu.async_copy(feat_ref.at[sidx], featbuf, sem1)
        pltpu.sync_copy(ee_ref.at[r], eebuf)
        cp1.wait()

        def _edge(j, _):
            a = eebuf[j, :]
            for h in range(_H):
                featbuf[j, pl.ds(16 * h, 16)] = (
                    featbuf[j, pl.ds(16 * h, 16)] * a[h])
            return 0
        lax.fori_loop(0, _CHUNK, _edge, 0)
        pltpu.sync_copy(featbuf, osh.at[didx], add=True)
        return 0
    lax.fori_loop(0, nb, _chunk, 0)

    plsc.subcore_barrier()
    for off, sz in _SUB_CHUNKS:
        o = pl.multiple_of(base + off, 8)
        pltpu.sync_copy(osh.at[pl.ds(o, sz)], op_ref.at[cid, pl.ds(o, sz)])

    @pl.when(sid == _NS - 1)
    def _xt():
        pltpu.sync_copy(osh.at[pl.ds(_TAIL_OFF, _TAIL)],
                        op_ref.at[cid, pl.ds(_TAIL_OFF, _TAIL)])


def _p2(src2, dst2, feat, ee):
    mesh = plsc.VectorSubcoreMesh(core_axis_name="c", subcore_axis_name="s")
    f = pl.kernel(
        _p2_body,
        out_type=jax.ShapeDtypeStruct((_NC, _N, _D), jnp.float32),
        mesh=mesh,
        scratch_types=[
            pltpu.VMEM((_CHUNK,), jnp.int32),
            pltpu.VMEM((_CHUNK,), jnp.int32),
            pltpu.VMEM((_CHUNK, _D), jnp.float32),
            pltpu.VMEM((_CHUNK, 16), jnp.float32),
            pltpu.VMEM_SHARED((_N, _D), jnp.float32),
            pltpu.SemaphoreType.DMA,
        ],
    )
    return f(src2, dst2, feat, ee)


def _ka_body(op0_ref, op1_ref, dp0_ref, dp1_ref, b0_ref, b1_ref,
             wp1_ref, bp1_ref, wp2_ref,
             z0_ref, z1_ref, w_ref):
    lane = lax.broadcasted_iota(jnp.int32, (8, _D), 1)
    row = lax.broadcasted_iota(jnp.int32, (8, _D), 0)
    eh = jnp.where(lane // _OUT == row, 1.0, 0.0)

    d0 = dp0_ref[0] + dp0_ref[1]
    d1 = dp1_ref[0] + dp1_ref[1]
    r0 = 1.0 / (d0[:, 0:8] + 1e-30)
    r1 = 1.0 / (d1[:, 0:8] + 1e-30)
    s0 = jnp.dot(r0, eh, preferred_element_type=jnp.float32)
    s1 = jnp.dot(r1, eh, preferred_element_type=jnp.float32)
    z0 = (op0_ref[0] + op0_ref[1]) * s0 + b0_ref[...]
    z1 = (op1_ref[0] + op1_ref[1]) * s1 + b1_ref[...]
    z0_ref[...] = z0
    z1_ref[...] = z1
    t0 = jnp.tanh(jnp.dot(z0, wp1_ref[...], preferred_element_type=jnp.float32)
                  + bp1_ref[...])
    t1 = jnp.tanh(jnp.dot(z1, wp1_ref[...], preferred_element_type=jnp.float32)
                  + bp1_ref[...])
    w0 = jnp.sum(t0 * wp2_ref[...], axis=1, keepdims=True)
    w1 = jnp.sum(t1 * wp2_ref[...], axis=1, keepdims=True)
    w_ref[...] = jnp.concatenate([w0, w1], axis=1)


def _ka(op0, op1, dp0, dp1, b0r, b1r, wp1, bp1r, wp2r):
    blk = _BLK
    return pl.pallas_call(
        _ka_body,
        grid=(_GRID,),
        in_specs=[
            pl.BlockSpec((_NC, blk, _D), lambda i: (0, i, 0)),
            pl.BlockSpec((_NC, blk, _D), lambda i: (0, i, 0)),
            pl.BlockSpec((_NC, blk, 16), lambda i: (0, i, 0)),
            pl.BlockSpec((_NC, blk, 16), lambda i: (0, i, 0)),
            pl.BlockSpec((1, _D), lambda i: (0, 0)),
            pl.BlockSpec((1, _D), lambda i: (0, 0)),
            pl.BlockSpec((_D, _HID), lambda i: (0, 0)),
            pl.BlockSpec((1, _HID), lambda i: (0, 0)),
            pl.BlockSpec((1, _HID), lambda i: (0, 0)),
        ],
        out_specs=[
            pl.BlockSpec((blk, _D), lambda i: (i, 0)),
            pl.BlockSpec((blk, _D), lambda i: (i, 0)),
            pl.BlockSpec((blk, 2), lambda i: (i, 0)),
        ],
        out_shape=[
            jax.ShapeDtypeStruct((_N, _D), jnp.float32),
            jax.ShapeDtypeStruct((_N, _D), jnp.float32),
            jax.ShapeDtypeStruct((_N, 2), jnp.float32),
        ],
    )(op0, op1, dp0, dp1, b0r, b1r, wp1, bp1r, wp2r)


def _kb_body(z0_ref, z1_ref, w_ref, out_ref):
    w = w_ref[...]
    s0 = jnp.sum(w[:, 0:1]) / _N
    s1 = jnp.sum(w[:, 1:2]) / _N
    m = jnp.maximum(s0, s1)
    e0 = jnp.exp(s0 - m)
    e1 = jnp.exp(s1 - m)
    beta0 = e0 / (e0 + e1)
    beta1 = e1 / (e0 + e1)
    out_ref[...] = beta0 * z0_ref[...] + beta1 * z1_ref[...]


def _kb(z0, z1, w):
    blk = _BLK
    return pl.pallas_call(
        _kb_body,
        grid=(_GRID,),
        in_specs=[
            pl.BlockSpec((blk, _D), lambda i: (i, 0)),
            pl.BlockSpec((blk, _D), lambda i: (i, 0)),
            pl.BlockSpec((_N, 2), lambda i: (0, 0)),
        ],
        out_specs=pl.BlockSpec((blk, _D), lambda i: (i, 0)),
        out_shape=jax.ShapeDtypeStruct((_N, _D), jnp.float32),
    )(z0, z1, w)


def _attn_mats(attn_l, attn_r):
    k = jnp.arange(_D)
    hp = jnp.arange(16)
    mask = (k[:, None] // _OUT) == (hp[None, :] % _H)
    ml = jnp.where(mask, attn_l.reshape(_D)[:, None], 0.0)
    mr = jnp.where(mask, attn_r.reshape(_D)[:, None], 0.0)
    return ml.astype(jnp.float32), mr.astype(jnp.float32)


def kernel(x, edge_index_0, edge_index_1, W0, attn_l0, attn_r0, b0,
           W1, attn_l1, attn_r1, b1, Wp1, bp1, Wp2):
    src0 = edge_index_0[0].reshape(_R, _CHUNK)
    dst0 = edge_index_0[1].reshape(_R, _CHUNK)
    src1 = edge_index_1[0].reshape(_R, _CHUNK)
    dst1 = edge_index_1[1].reshape(_R, _CHUNK)

    ml0, mr0 = _attn_mats(attn_l0, attn_r0)
    ml1, mr1 = _attn_mats(attn_l1, attn_r1)

    feat0, feat1, tl0, tr0, tl1, tr1, c = _pre(x, W0, W1, ml0, mr0, ml1, mr1)

    ee0, dp0 = _p1(src0, dst0, tl0, tr0, c[0:1])
    ee1, dp1 = _p1(src1, dst1, tl1, tr1, c[1:2])

    op0 = _p2(src0, dst0, feat0, ee0)
    op1 = _p2(src1, dst1, feat1, ee1)

    z0, z1, w = _ka(op0, op1, dp0, dp1, b0.reshape(1, _D), b1.reshape(1, _D),
                    Wp1, bp1.reshape(1, _HID), Wp2.reshape(1, _HID))
    return _kb(z0, z1, w)


# speedup vs baseline: 88.7149x; 1.3195x over previous
import jax
import jax.numpy as jnp
from jax import lax
from jax.experimental import pallas as pl
from jax.experimental.pallas import tpu as pltpu
from jax.experimental.pallas import tpu_sc as plsc

_N = 10000
_E = 320000
_IN = 128
_H = 8
_OUT = 16
_D = _H * _OUT
_HID = 128
_CHUNK = 128
_R = _E // _CHUNK
_NC = 2
_NS = 16
_SUB_BASE = 624
_SUB_CHUNKS = ((0, 128), (128, 128), (256, 128), (384, 128), (512, 112))
_TAIL_OFF = _SUB_BASE * _NS
_TAIL = _N - _TAIL_OFF
_LEAK = 0.2
_BLK = 2000
_GRID = _N // _BLK
_NB_BASE = _R // _NS
_NB_EXTRA = _R - _NB_BASE * _NS
_NPAIR = (_NB_BASE + _NB_EXTRA + 1) // 2


def _pre_body(x_ref, w_ref, ml_ref, mr_ref,
              feat_ref, tl_ref, tr_ref, c_ref, acc_ref):
    i = pl.program_id(1)
    x = x_ref[...]
    f = jnp.dot(x, w_ref[0], preferred_element_type=jnp.float32)
    feat_ref[0] = f
    tl = jnp.dot(f, ml_ref[0], preferred_element_type=jnp.float32)
    tr = jnp.dot(f, mr_ref[0], preferred_element_type=jnp.float32)
    tl_ref[0] = tl
    tr_ref[0] = tr
    for row, t in enumerate((tl, tr)):
        m = jnp.max(t, axis=0)
        prev = acc_ref[row, :]
        acc_ref[row, :] = jnp.where(i == 0, m, jnp.maximum(prev, m))
    zero = jnp.zeros((16,), jnp.float32)
    c_ref[0, 0, :] = jnp.maximum(zero, acc_ref[0, :] + acc_ref[1, :])


def _pre(x, w, ml, mr):
    blk = _BLK
    return pl.pallas_call(
        _pre_body,
        grid=(_NC, _GRID),
        in_specs=[
            pl.BlockSpec((blk, _IN), lambda p, i: (i, 0)),
            pl.BlockSpec((1, _IN, _D), lambda p, i: (p, 0, 0)),
            pl.BlockSpec((1, _D, 16), lambda p, i: (p, 0, 0)),
            pl.BlockSpec((1, _D, 16), lambda p, i: (p, 0, 0)),
        ],
        out_specs=[
            pl.BlockSpec((1, blk, _D), lambda p, i: (p, i, 0)),
            pl.BlockSpec((1, blk, 16), lambda p, i: (p, i, 0)),
            pl.BlockSpec((1, blk, 16), lambda p, i: (p, i, 0)),
            pl.BlockSpec((1, 1, 16), lambda p, i: (p, 0, 0)),
        ],
        out_shape=[
            jax.ShapeDtypeStruct((_NC, _N, _D), jnp.float32),
            jax.ShapeDtypeStruct((_NC, _N, 16), jnp.float32),
            jax.ShapeDtypeStruct((_NC, _N, 16), jnp.float32),
            jax.ShapeDtypeStruct((_NC, 1, 16), jnp.float32),
        ],
        scratch_shapes=[pltpu.VMEM((2, 16), jnp.float32)],
    )(x, w, ml, mr)


def _zero_shared(zbuf, sh, sid, width):
    base = pl.multiple_of(sid * _SUB_BASE, 8)
    for off, sz in _SUB_CHUNKS:
        pltpu.sync_copy(zbuf.at[pl.ds(0, sz)],
                        sh.at[pl.ds(pl.multiple_of(base + off, 8), sz)])

    @pl.when(sid == _NS - 1)
    def _zt():
        pltpu.sync_copy(zbuf.at[pl.ds(0, _TAIL)], sh.at[pl.ds(_TAIL_OFF, _TAIL)])


def _export_shared(sh, out2d_at_cid, sid):
    base = pl.multiple_of(sid * _SUB_BASE, 8)
    for off, sz in _SUB_CHUNKS:
        o = pl.multiple_of(base + off, 8)
        pltpu.sync_copy(sh.at[pl.ds(o, sz)], out2d_at_cid(o, sz))

    @pl.when(sid == _NS - 1)
    def _xt():
        pltpu.sync_copy(sh.at[pl.ds(_TAIL_OFF, _TAIL)],
                        out2d_at_cid(_TAIL_OFF, _TAIL))


def _p1_body(src_ref, dst_ref, tl_ref, tr_ref, c_ref,
             ee_ref, dp_ref,
             sx0, dx0, dg0, sx1, dx1, dg1,
             ab0, bb0, ab1, bb1, eb0, eb1,
             cbuf, dsh, sa0, sb0, sa1, sb1):
    cid = lax.axis_index("c")
    sid = lax.axis_index("s")

    def _z(i, _):
        eb0[i, :] = jnp.zeros((16,), jnp.float32)
        return 0
    lax.fori_loop(0, _CHUNK, _z, 0)
    _zero_shared(eb0, dsh, sid, 16)
    plsc.subcore_barrier()

    pltpu.sync_copy(c_ref, cbuf)
    cvec = cbuf[cid, :]
    noff = cid * _N

    nb = _NB_BASE + jnp.where(sid < _NB_EXTRA, 1, 0)

    def _issue(ci, sx, dx, dg, ab, bb, sa, sb):
        r = cid * _R + sid + _NS * ci
        pltpu.sync_copy(src_ref.at[r], sx)
        pltpu.sync_copy(dst_ref.at[r], dx)
        for k in range(_CHUNK // 16):
            s = sx[pl.ds(16 * k, 16)] + noff
            sx[pl.ds(16 * k, 16)] = s
            dg[pl.ds(16 * k, 16)] = dx[pl.ds(16 * k, 16)] + noff
        pltpu.async_copy(tl_ref.at[sx], ab, sa)
        pltpu.async_copy(tr_ref.at[dg], bb, sb)

    def _wait(ab, bb, sa, sb, sx, dg):
        pltpu.make_async_copy(tl_ref.at[sx], ab, sa).wait()
        pltpu.make_async_copy(tr_ref.at[dg], bb, sb).wait()

    def _compute(ci, ab, bb, eb, dx):
        def _edge(j, _):
            for jj in (2 * j, 2 * j + 1):
                e = ab[jj, :] + bb[jj, :]
                e = jnp.maximum(e, _LEAK * e) - cvec
                eb[jj, :] = jnp.exp(e)
            return 0
        lax.fori_loop(0, _CHUNK // 2, _edge, 0)
        r = cid * _R + sid + _NS * ci
        pltpu.sync_copy(eb, ee_ref.at[r])
        pltpu.sync_copy(eb, dsh.at[dx], add=True)

    _issue(0, sx0, dx0, dg0, ab0, bb0, sa0, sb0)

    def _pair(i, _):
        c0 = 2 * i
        c1 = 2 * i + 1
        c2 = 2 * i + 2

        @pl.when(c1 < nb)
        def _i1():
            _issue(c1, sx1, dx1, dg1, ab1, bb1, sa1, sb1)

        @pl.when(c0 < nb)
        def _c0():
            _wait(ab0, bb0, sa0, sb0, sx0, dg0)
            _compute(c0, ab0, bb0, eb0, dx0)

        @pl.when(c2 < nb)
        def _i2():
            _issue(c2, sx0, dx0, dg0, ab0, bb0, sa0, sb0)

        @pl.when(c1 < nb)
        def _c1():
            _wait(ab1, bb1, sa1, sb1, sx1, dg1)
            _compute(c1, ab1, bb1, eb1, dx1)
        return 0
    lax.fori_loop(0, _NPAIR, _pair, 0)

    plsc.subcore_barrier()
    _export_shared(dsh, lambda o, sz: dp_ref.at[cid, pl.ds(o, sz)], sid)


def _p1(src2, dst2, tl, tr, c):
    mesh = plsc.VectorSubcoreMesh(core_axis_name="c", subcore_axis_name="s")
    f = pl.kernel(
        _p1_body,
        out_type=[
            jax.ShapeDtypeStruct((_NC * _R, _CHUNK, 16), jnp.float32),
            jax.ShapeDtypeStruct((_NC, _N, 16), jnp.float32),
        ],
        mesh=mesh,
        compiler_params=pltpu.CompilerParams(use_tc_tiling_on_sc=False),
        scratch_types=[
            pltpu.VMEM((_CHUNK,), jnp.int32),
            pltpu.VMEM((_CHUNK,), jnp.int32),
            pltpu.VMEM((_CHUNK,), jnp.int32),
            pltpu.VMEM((_CHUNK,), jnp.int32),
            pltpu.VMEM((_CHUNK,), jnp.int32),
            pltpu.VMEM((_CHUNK,), jnp.int32),
            pltpu.VMEM((_CHUNK, 16), jnp.float32),
            pltpu.VMEM((_CHUNK, 16), jnp.float32),
            pltpu.VMEM((_CHUNK, 16), jnp.float32),
            pltpu.VMEM((_CHUNK, 16), jnp.float32),
            pltpu.VMEM((_CHUNK, 16), jnp.float32),
            pltpu.VMEM((_CHUNK, 16), jnp.float32),
            pltpu.VMEM((_NC, 16), jnp.float32),
            pltpu.VMEM_SHARED((_N, 16), jnp.float32),
            pltpu.SemaphoreType.DMA,
            pltpu.SemaphoreType.DMA,
            pltpu.SemaphoreType.DMA,
            pltpu.SemaphoreType.DMA,
        ],
    )
    return f(src2, dst2, tl, tr, c)


def _p2_body(src_ref, dst_ref, feat_ref, ee_ref,
             op_ref,
             sx0, dx0, sx1, dx1, fb0, fb1, eb0, eb1, osh, sa0, sa1):
    cid = lax.axis_index("c")
    sid = lax.axis_index("s")

    def _z(i, _):
        for h in range(_H):
            fb0[i, pl.ds(16 * h, 16)] = jnp.zeros((16,), jnp.float32)
        return 0
    lax.fori_loop(0, _CHUNK, _z, 0)
    _zero_shared(fb0, osh, sid, _D)
    plsc.subcore_barrier()

    noff = cid * _N
    nb = _NB_BASE + jnp.where(sid < _NB_EXTRA, 1, 0)

    def _issue(ci, sx, dx, fb, eb, sa):
        r = cid * _R + sid + _NS * ci
        pltpu.sync_copy(src_ref.at[r], sx)
        pltpu.sync_copy(dst_ref.at[r], dx)
        for k in range(_CHUNK // 16):
            sx[pl.ds(16 * k, 16)] = sx[pl.ds(16 * k, 16)] + noff
        pltpu.async_copy(feat_ref.at[sx], fb, sa)
        pltpu.sync_copy(ee_ref.at[r], eb)

    def _compute(ab_sx, fb, eb, dx, sa):
        pltpu.make_async_copy(feat_ref.at[ab_sx], fb, sa).wait()

        def _edge(j, _):
            for jj in (2 * j, 2 * j + 1):
                a = eb[jj, :]
                for h in range(_H):
                    fb[jj, pl.ds(16 * h, 16)] = (
                        fb[jj, pl.ds(16 * h, 16)] * a[h])
            return 0
        lax.fori_loop(0, _CHUNK // 2, _edge, 0)
        pltpu.sync_copy(fb, osh.at[dx], add=True)

    _issue(0, sx0, dx0, fb0, eb0, sa0)

    def _pair(i, _):
        c0 = 2 * i
        c1 = 2 * i + 1
        c2 = 2 * i + 2

        @pl.when(c1 < nb)
        def _i1():
            _issue(c1, sx1, dx1, fb1, eb1, sa1)

        @pl.when(c0 < nb)
        def _c0():
            _compute(sx0, fb0, eb0, dx0, sa0)

        @pl.when(c2 < nb)
        def _i2():
            _issue(c2, sx0, dx0, fb0, eb0, sa0)

        @pl.when(c1 < nb)
        def _c1():
            _compute(sx1, fb1, eb1, dx1, sa1)
        return 0
    lax.fori_loop(0, _NPAIR, _pair, 0)

    plsc.subcore_barrier()
    _export_shared(osh, lambda o, sz: op_ref.at[cid, pl.ds(o, sz)], sid)


def _p2(src2, dst2, feat, ee):
    mesh = plsc.VectorSubcoreMesh(core_axis_name="c", subcore_axis_name="s")
    f = pl.kernel(
        _p2_body,
        out_type=jax.ShapeDtypeStruct((_NC, _N, _D), jnp.float32),
        mesh=mesh,
        compiler_params=pltpu.CompilerParams(use_tc_tiling_on_sc=False),
        scratch_types=[
            pltpu.VMEM((_CHUNK,), jnp.int32),
            pltpu.VMEM((_CHUNK,), jnp.int32),
            pltpu.VMEM((_CHUNK,), jnp.int32),
            pltpu.VMEM((_CHUNK,), jnp.int32),
            pltpu.VMEM((_CHUNK, _D), jnp.float32),
            pltpu.VMEM((_CHUNK, _D), jnp.float32),
            pltpu.VMEM((_CHUNK, 16), jnp.float32),
            pltpu.VMEM((_CHUNK, 16), jnp.float32),
            pltpu.VMEM_SHARED((_N, _D), jnp.float32),
            pltpu.SemaphoreType.DMA,
            pltpu.SemaphoreType.DMA,
        ],
    )
    return f(src2, dst2, feat, ee)


def _ka_body(op_ref, dp_ref, bb_ref, wp1_ref, bp1_ref, wp2_ref,
             z0_ref, z1_ref, w_ref):
    lane = lax.broadcasted_iota(jnp.int32, (8, _D), 1)
    row = lax.broadcasted_iota(jnp.int32, (8, _D), 0)
    eh = jnp.where(lane // _OUT == row, 1.0, 0.0)

    r0 = 1.0 / (dp_ref[0][:, 0:8] + 1e-30)
    r1 = 1.0 / (dp_ref[1][:, 0:8] + 1e-30)
    s0 = jnp.dot(r0, eh, preferred_element_type=jnp.float32)
    s1 = jnp.dot(r1, eh, preferred_element_type=jnp.float32)
    z0 = op_ref[0] * s0 + bb_ref[0:1, :]
    z1 = op_ref[1] * s1 + bb_ref[1:2, :]
    z0_ref[...] = z0
    z1_ref[...] = z1
    t0 = jnp.tanh(jnp.dot(z0, wp1_ref[...], preferred_element_type=jnp.float32)
                  + bp1_ref[...])
    t1 = jnp.tanh(jnp.dot(z1, wp1_ref[...], preferred_element_type=jnp.float32)
                  + bp1_ref[...])
    w0 = jnp.sum(t0 * wp2_ref[...], axis=1, keepdims=True)
    w1 = jnp.sum(t1 * wp2_ref[...], axis=1, keepdims=True)
    w_ref[...] = jnp.concatenate([w0, w1], axis=1)


def _ka(op, dp, bb, wp1, bp1r, wp2r):
    blk = _BLK
    return pl.pallas_call(
        _ka_body,
        grid=(_GRID,),
        in_specs=[
            pl.BlockSpec((_NC, blk, _D), lambda i: (0, i, 0)),
            pl.BlockSpec((_NC, blk, 16), lambda i: (0, i, 0)),
            pl.BlockSpec((_NC, _D), lambda i: (0, 0)),
            pl.BlockSpec((_D, _HID), lambda i: (0, 0)),
            pl.BlockSpec((1, _HID), lambda i: (0, 0)),
            pl.BlockSpec((1, _HID), lambda i: (0, 0)),
        ],
        out_specs=[
            pl.BlockSpec((blk, _D), lambda i: (i, 0)),
            pl.BlockSpec((blk, _D), lambda i: (i, 0)),
            pl.BlockSpec((blk, 2), lambda i: (i, 0)),
        ],
        out_shape=[
            jax.ShapeDtypeStruct((_N, _D), jnp.float32),
            jax.ShapeDtypeStruct((_N, _D), jnp.float32),
            jax.ShapeDtypeStruct((_N, 2), jnp.float32),
        ],
    )(op, dp, bb, wp1, bp1r, wp2r)


def _kb_body(z0_ref, z1_ref, w_ref, out_ref):
    w = w_ref[...]
    s0 = jnp.sum(w[:, 0:1]) / _N
    s1 = jnp.sum(w[:, 1:2]) / _N
    m = jnp.maximum(s0, s1)
    e0 = jnp.exp(s0 - m)
    e1 = jnp.exp(s1 - m)
    beta0 = e0 / (e0 + e1)
    beta1 = e1 / (e0 + e1)
    out_ref[...] = beta0 * z0_ref[...] + beta1 * z1_ref[...]


def _kb(z0, z1, w):
    blk = _BLK
    return pl.pallas_call(
        _kb_body,
        grid=(_GRID,),
        in_specs=[
            pl.BlockSpec((blk, _D), lambda i: (i, 0)),
            pl.BlockSpec((blk, _D), lambda i: (i, 0)),
            pl.BlockSpec((_N, 2), lambda i: (0, 0)),
        ],
        out_specs=pl.BlockSpec((blk, _D), lambda i: (i, 0)),
        out_shape=jax.ShapeDtypeStruct((_N, _D), jnp.float32),
    )(z0, z1, w)


def _attn_mats(attn_l, attn_r):
    k = jnp.arange(_D)
    hp = jnp.arange(16)
    mask = (k[:, None] // _OUT) == (hp[None, :] % _H)
    ml = jnp.where(mask, attn_l.reshape(_D)[:, None], 0.0)
    mr = jnp.where(mask, attn_r.reshape(_D)[:, None], 0.0)
    return ml.astype(jnp.float32), mr.astype(jnp.float32)


def kernel(x, edge_index_0, edge_index_1, W0, attn_l0, attn_r0, b0,
           W1, attn_l1, attn_r1, b1, Wp1, bp1, Wp2):
    src2 = jnp.stack([edge_index_0[0], edge_index_1[0]]).reshape(_NC * _R, _CHUNK)
    dst2 = jnp.stack([edge_index_0[1], edge_index_1[1]]).reshape(_NC * _R, _CHUNK)

    ml0, mr0 = _attn_mats(attn_l0, attn_r0)
    ml1, mr1 = _attn_mats(attn_l1, attn_r1)
    w = jnp.stack([W0, W1])
    ml = jnp.stack([ml0, ml1])
    mr = jnp.stack([mr0, mr1])
    bb = jnp.stack([b0, b1])

    feat, tl, tr, c = _pre(x, w, ml, mr)

    ee, dp = _p1(src2, dst2, tl.reshape(_NC * _N, 16),
                 tr.reshape(_NC * _N, 16), c.reshape(_NC, 16))
    op = _p2(src2, dst2, feat.reshape(_NC * _N, _D), ee)

    z0, z1, wsem = _ka(op, dp, bb, Wp1, bp1.reshape(1, _HID),
                       Wp2.reshape(1, _HID))
    return _kb(z0, z1, wsem)


# speedup vs baseline: 88.7811x; 1.0007x over previous
import jax
import jax.numpy as jnp
from jax import lax
from jax.experimental import pallas as pl
from jax.experimental.pallas import tpu as pltpu
from jax.experimental.pallas import tpu_sc as plsc

_N = 10000
_E = 320000
_IN = 128
_H = 8
_OUT = 16
_D = _H * _OUT
_HID = 128
_CHUNK = 128
_R = _E // _CHUNK
_NC = 2
_NS = 16
_SUB_BASE = 624
_SUB_CHUNKS = ((0, 128), (128, 128), (256, 128), (384, 128), (512, 112))
_TAIL_OFF = _SUB_BASE * _NS
_TAIL = _N - _TAIL_OFF
_LEAK = 0.2
_BLK = 2000
_GRID = _N // _BLK
_NB_BASE = _R // _NS
_NB_EXTRA = _R - _NB_BASE * _NS
_NPAIR = (_NB_BASE + _NB_EXTRA + 1) // 2


def _pre_body(x_ref, w_ref, ml_ref, mr_ref,
              feat_ref, tl_ref, tr_ref, c_ref, acc_ref):
    i = pl.program_id(1)
    x = x_ref[...]
    f = jnp.dot(x, w_ref[0], preferred_element_type=jnp.float32)
    feat_ref[0] = f
    tl = jnp.dot(f, ml_ref[0], preferred_element_type=jnp.float32)
    tr = jnp.dot(f, mr_ref[0], preferred_element_type=jnp.float32)
    tl_ref[0] = tl
    tr_ref[0] = tr
    for row, t in enumerate((tl, tr)):
        m = jnp.max(t, axis=0)
        prev = acc_ref[row, :]
        acc_ref[row, :] = jnp.where(i == 0, m, jnp.maximum(prev, m))
    zero = jnp.zeros((16,), jnp.float32)
    c_ref[0, 0, :] = jnp.maximum(zero, acc_ref[0, :] + acc_ref[1, :])


def _pre(x, w, ml, mr):
    blk = _BLK
    return pl.pallas_call(
        _pre_body,
        grid=(_NC, _GRID),
        in_specs=[
            pl.BlockSpec((blk, _IN), lambda p, i: (i, 0)),
            pl.BlockSpec((1, _IN, _D), lambda p, i: (p, 0, 0)),
            pl.BlockSpec((1, _D, 16), lambda p, i: (p, 0, 0)),
            pl.BlockSpec((1, _D, 16), lambda p, i: (p, 0, 0)),
        ],
        out_specs=[
            pl.BlockSpec((1, blk, _D), lambda p, i: (p, i, 0)),
            pl.BlockSpec((1, blk, 16), lambda p, i: (p, i, 0)),
            pl.BlockSpec((1, blk, 16), lambda p, i: (p, i, 0)),
            pl.BlockSpec((1, 1, 16), lambda p, i: (p, 0, 0)),
        ],
        out_shape=[
            jax.ShapeDtypeStruct((_NC, _N, _D), jnp.float32),
            jax.ShapeDtypeStruct((_NC, _N, 16), jnp.float32),
            jax.ShapeDtypeStruct((_NC, _N, 16), jnp.float32),
            jax.ShapeDtypeStruct((_NC, 1, 16), jnp.float32),
        ],
        scratch_shapes=[pltpu.VMEM((2, 16), jnp.float32)],
    )(x, w, ml, mr)


def _zero_shared(zbuf, sh, sid, width):
    base = pl.multiple_of(sid * _SUB_BASE, 8)
    for off, sz in _SUB_CHUNKS:
        pltpu.sync_copy(zbuf.at[pl.ds(0, sz)],
                        sh.at[pl.ds(pl.multiple_of(base + off, 8), sz)])

    @pl.when(sid == _NS - 1)
    def _zt():
        pltpu.sync_copy(zbuf.at[pl.ds(0, _TAIL)], sh.at[pl.ds(_TAIL_OFF, _TAIL)])


def _export_shared(sh, out2d_at_cid, sid):
    base = pl.multiple_of(sid * _SUB_BASE, 8)
    for off, sz in _SUB_CHUNKS:
        o = pl.multiple_of(base + off, 8)
        pltpu.sync_copy(sh.at[pl.ds(o, sz)], out2d_at_cid(o, sz))

    @pl.when(sid == _NS - 1)
    def _xt():
        pltpu.sync_copy(sh.at[pl.ds(_TAIL_OFF, _TAIL)],
                        out2d_at_cid(_TAIL_OFF, _TAIL))


def _p1_body(src_ref, dst_ref, tl_ref, tr_ref, c_ref,
             ee_ref, dp_ref,
             sx0, dx0, dg0, sx1, dx1, dg1,
             ab0, bb0, ab1, bb1, eb0, eb1,
             cbuf, dsh, sa0, sb0, sa1, sb1):
    cid = lax.axis_index("c")
    sid = lax.axis_index("s")

    def _z(i, _):
        eb0[i, :] = jnp.zeros((16,), jnp.float32)
        return 0
    lax.fori_loop(0, _CHUNK, _z, 0)
    _zero_shared(eb0, dsh, sid, 16)
    plsc.subcore_barrier()

    pltpu.sync_copy(c_ref, cbuf)
    cvec = cbuf[cid, :]
    noff = cid * _N

    nb = _NB_BASE + jnp.where(sid < _NB_EXTRA, 1, 0)

    def _issue(ci, sx, dx, dg, ab, bb, sa, sb):
        r = cid * _R + sid + _NS * ci
        pltpu.sync_copy(src_ref.at[r], sx)
        pltpu.sync_copy(dst_ref.at[r], dx)
        for k in range(_CHUNK // 16):
            sx[pl.ds(16 * k, 16)] = sx[pl.ds(16 * k, 16)] + noff
            dg[pl.ds(16 * k, 16)] = dx[pl.ds(16 * k, 16)] + noff
        pltpu.async_copy(tl_ref.at[sx], ab, sa)
        pltpu.async_copy(tr_ref.at[dg], bb, sb)

    def _wait(ab, bb, sa, sb):
        pltpu.make_async_copy(tl_ref.at[pl.ds(0, _CHUNK)], ab, sa).wait()
        pltpu.make_async_copy(tr_ref.at[pl.ds(0, _CHUNK)], bb, sb).wait()

    def _compute(ci, ab, bb, eb, dx):
        def _edge(j, _):
            for jj in (2 * j, 2 * j + 1):
                e = ab[jj, :] + bb[jj, :]
                e = jnp.maximum(e, _LEAK * e) - cvec
                eb[jj, :] = jnp.exp(e)
            return 0
        lax.fori_loop(0, _CHUNK // 2, _edge, 0)
        r = cid * _R + sid + _NS * ci
        pltpu.sync_copy(eb, ee_ref.at[r])
        pltpu.sync_copy(eb, dsh.at[dx], add=True)

    _issue(0, sx0, dx0, dg0, ab0, bb0, sa0, sb0)

    def _pair(i, _):
        c0 = 2 * i
        c1 = 2 * i + 1
        c2 = 2 * i + 2

        @pl.when(c1 < nb)
        def _i1():
            _issue(c1, sx1, dx1, dg1, ab1, bb1, sa1, sb1)

        @pl.when(c0 < nb)
        def _c0():
            _wait(ab0, bb0, sa0, sb0)
            _compute(c0, ab0, bb0, eb0, dx0)

        @pl.when(c2 < nb)
        def _i2():
            _issue(c2, sx0, dx0, dg0, ab0, bb0, sa0, sb0)

        @pl.when(c1 < nb)
        def _c1():
            _wait(ab1, bb1, sa1, sb1)
            _compute(c1, ab1, bb1, eb1, dx1)
        return 0
    lax.fori_loop(0, _NPAIR, _pair, 0)

    plsc.subcore_barrier()
    _export_shared(dsh, lambda o, sz: dp_ref.at[cid, pl.ds(o, sz)], sid)


def _p1(src2, dst2, tl, tr, c):
    mesh = plsc.VectorSubcoreMesh(core_axis_name="c", subcore_axis_name="s")
    f = pl.kernel(
        _p1_body,
        out_type=[
            jax.ShapeDtypeStruct((_NC * _R, _CHUNK, 16), jnp.float32),
            jax.ShapeDtypeStruct((_NC, _N, 16), jnp.float32),
        ],
        mesh=mesh,
        compiler_params=pltpu.CompilerParams(use_tc_tiling_on_sc=False),
        scratch_types=[
            pltpu.VMEM((_CHUNK,), jnp.int32),
            pltpu.VMEM((_CHUNK,), jnp.int32),
            pltpu.VMEM((_CHUNK,), jnp.int32),
            pltpu.VMEM((_CHUNK,), jnp.int32),
            pltpu.VMEM((_CHUNK,), jnp.int32),
            pltpu.VMEM((_CHUNK,), jnp.int32),
            pltpu.VMEM((_CHUNK, 16), jnp.float32),
            pltpu.VMEM((_CHUNK, 16), jnp.float32),
            pltpu.VMEM((_CHUNK, 16), jnp.float32),
            pltpu.VMEM((_CHUNK, 16), jnp.float32),
            pltpu.VMEM((_CHUNK, 16), jnp.float32),
            pltpu.VMEM((_CHUNK, 16), jnp.float32),
            pltpu.VMEM((_NC, 16), jnp.float32),
            pltpu.VMEM_SHARED((_N, 16), jnp.float32),
            pltpu.SemaphoreType.DMA,
            pltpu.SemaphoreType.DMA,
            pltpu.SemaphoreType.DMA,
            pltpu.SemaphoreType.DMA,
        ],
    )
    return f(src2, dst2, tl, tr, c)


def _p2_body(src_ref, dst_ref, feat_ref, ee_ref,
             op_ref,
             sx0, dx0, sx1, dx1, fb0, fb1, eb0, eb1, osh,
             sa0, sa1, so0, so1):
    cid = lax.axis_index("c")
    sid = lax.axis_index("s")

    def _z(i, _):
        for h in range(_H):
            fb0[i, pl.ds(16 * h, 16)] = jnp.zeros((16,), jnp.float32)
        return 0
    lax.fori_loop(0, _CHUNK, _z, 0)
    _zero_shared(fb0, osh, sid, _D)
    plsc.subcore_barrier()

    noff = cid * _N
    nb = _NB_BASE + jnp.where(sid < _NB_EXTRA, 1, 0)

    def _issue(ci, first, sx, dx, fb, eb, sa, so):
        @pl.when(jnp.logical_not(first))
        def _dr():
            pltpu.make_async_copy(fb, osh.at[pl.ds(0, _CHUNK)], so).wait()
        r = cid * _R + sid + _NS * ci
        pltpu.sync_copy(src_ref.at[r], sx)
        pltpu.sync_copy(dst_ref.at[r], dx)
        for k in range(_CHUNK // 16):
            sx[pl.ds(16 * k, 16)] = sx[pl.ds(16 * k, 16)] + noff
        pltpu.async_copy(feat_ref.at[sx], fb, sa)
        pltpu.sync_copy(ee_ref.at[r], eb)

    def _compute(fb, eb, dx, sa, so):
        pltpu.make_async_copy(feat_ref.at[pl.ds(0, _CHUNK)], fb, sa).wait()

        def _edge(j, _):
            for jj in (2 * j, 2 * j + 1):
                a = eb[jj, :]
                for h in range(_H):
                    fb[jj, pl.ds(16 * h, 16)] = (
                        fb[jj, pl.ds(16 * h, 16)] * a[h])
            return 0
        lax.fori_loop(0, _CHUNK // 2, _edge, 0)
        pltpu.async_copy(fb, osh.at[dx], so, add=True)

    _issue(0, True, sx0, dx0, fb0, eb0, sa0, so0)

    def _pair(i, _):
        c0 = 2 * i
        c1 = 2 * i + 1
        c2 = 2 * i + 2

        @pl.when(c1 < nb)
        def _i1():
            _issue(c1, i == 0, sx1, dx1, fb1, eb1, sa1, so1)

        @pl.when(c0 < nb)
        def _c0():
            _compute(fb0, eb0, dx0, sa0, so0)

        @pl.when(c2 < nb)
        def _i2():
            _issue(c2, False, sx0, dx0, fb0, eb0, sa0, so0)

        @pl.when(c1 < nb)
        def _c1():
            _compute(fb1, eb1, dx1, sa1, so1)
        return 0
    lax.fori_loop(0, _NPAIR, _pair, 0)

    @pl.when(nb >= 1)
    def _dr0():
        pltpu.make_async_copy(fb0, osh.at[pl.ds(0, _CHUNK)], so0).wait()

    @pl.when(nb >= 2)
    def _dr1():
        pltpu.make_async_copy(fb1, osh.at[pl.ds(0, _CHUNK)], so1).wait()
    plsc.subcore_barrier()
    _export_shared(osh, lambda o, sz: op_ref.at[cid, pl.ds(o, sz)], sid)


def _p2(src2, dst2, feat, ee):
    mesh = plsc.VectorSubcoreMesh(core_axis_name="c", subcore_axis_name="s")
    f = pl.kernel(
        _p2_body,
        out_type=jax.ShapeDtypeStruct((_NC, _N, _D), jnp.float32),
        mesh=mesh,
        compiler_params=pltpu.CompilerParams(use_tc_tiling_on_sc=False),
        scratch_types=[
            pltpu.VMEM((_CHUNK,), jnp.int32),
            pltpu.VMEM((_CHUNK,), jnp.int32),
            pltpu.VMEM((_CHUNK,), jnp.int32),
            pltpu.VMEM((_CHUNK,), jnp.int32),
            pltpu.VMEM((_CHUNK, _D), jnp.float32),
            pltpu.VMEM((_CHUNK, _D), jnp.float32),
            pltpu.VMEM((_CHUNK, 16), jnp.float32),
            pltpu.VMEM((_CHUNK, 16), jnp.float32),
            pltpu.VMEM_SHARED((_N, _D), jnp.float32),
            pltpu.SemaphoreType.DMA,
            pltpu.SemaphoreType.DMA,
            pltpu.SemaphoreType.DMA,
            pltpu.SemaphoreType.DMA,
        ],
    )
    return f(src2, dst2, feat, ee)


def _ka_body(op_ref, dp_ref, bb_ref, wp1_ref, bp1_ref, wp2_ref,
             z0_ref, z1_ref, w_ref):
    lane = lax.broadcasted_iota(jnp.int32, (8, _D), 1)
    row = lax.broadcasted_iota(jnp.int32, (8, _D), 0)
    eh = jnp.where(lane // _OUT == row, 1.0, 0.0)

    r0 = 1.0 / (dp_ref[0][:, 0:8] + 1e-30)
    r1 = 1.0 / (dp_ref[1][:, 0:8] + 1e-30)
    s0 = jnp.dot(r0, eh, preferred_element_type=jnp.float32)
    s1 = jnp.dot(r1, eh, preferred_element_type=jnp.float32)
    z0 = op_ref[0] * s0 + bb_ref[0:1, :]
    z1 = op_ref[1] * s1 + bb_ref[1:2, :]
    z0_ref[...] = z0
    z1_ref[...] = z1
    t0 = jnp.tanh(jnp.dot(z0, wp1_ref[...], preferred_element_type=jnp.float32)
                  + bp1_ref[...])
    t1 = jnp.tanh(jnp.dot(z1, wp1_ref[...], preferred_element_type=jnp.float32)
                  + bp1_ref[...])
    w0 = jnp.sum(t0 * wp2_ref[...], axis=1, keepdims=True)
    w1 = jnp.sum(t1 * wp2_ref[...], axis=1, keepdims=True)
    w_ref[...] = jnp.concatenate([w0, w1], axis=1)


def _ka(op, dp, bb, wp1, bp1r, wp2r):
    blk = _BLK
    return pl.pallas_call(
        _ka_body,
        grid=(_GRID,),
        in_specs=[
            pl.BlockSpec((_NC, blk, _D), lambda i: (0, i, 0)),
            pl.BlockSpec((_NC, blk, 16), lambda i: (0, i, 0)),
            pl.BlockSpec((_NC, _D), lambda i: (0, 0)),
            pl.BlockSpec((_D, _HID), lambda i: (0, 0)),
            pl.BlockSpec((1, _HID), lambda i: (0, 0)),
            pl.BlockSpec((1, _HID), lambda i: (0, 0)),
        ],
        out_specs=[
            pl.BlockSpec((blk, _D), lambda i: (i, 0)),
            pl.BlockSpec((blk, _D), lambda i: (i, 0)),
            pl.BlockSpec((blk, 2), lambda i: (i, 0)),
        ],
        out_shape=[
            jax.ShapeDtypeStruct((_N, _D), jnp.float32),
            jax.ShapeDtypeStruct((_N, _D), jnp.float32),
            jax.ShapeDtypeStruct((_N, 2), jnp.float32),
        ],
    )(op, dp, bb, wp1, bp1r, wp2r)


def _kb_body(z0_ref, z1_ref, w_ref, out_ref):
    w = w_ref[...]
    s0 = jnp.sum(w[:, 0:1]) / _N
    s1 = jnp.sum(w[:, 1:2]) / _N
    m = jnp.maximum(s0, s1)
    e0 = jnp.exp(s0 - m)
    e1 = jnp.exp(s1 - m)
    beta0 = e0 / (e0 + e1)
    beta1 = e1 / (e0 + e1)
    out_ref[...] = beta0 * z0_ref[...] + beta1 * z1_ref[...]


def _kb(z0, z1, w):
    blk = _BLK
    return pl.pallas_call(
        _kb_body,
        grid=(_GRID,),
        in_specs=[
            pl.BlockSpec((blk, _D), lambda i: (i, 0)),
            pl.BlockSpec((blk, _D), lambda i: (i, 0)),
            pl.BlockSpec((_N, 2), lambda i: (0, 0)),
        ],
        out_specs=pl.BlockSpec((blk, _D), lambda i: (i, 0)),
        out_shape=jax.ShapeDtypeStruct((_N, _D), jnp.float32),
    )(z0, z1, w)


def _attn_mats(attn_l, attn_r):
    k = jnp.arange(_D)
    hp = jnp.arange(16)
    mask = (k[:, None] // _OUT) == (hp[None, :] % _H)
    ml = jnp.where(mask, attn_l.reshape(_D)[:, None], 0.0)
    mr = jnp.where(mask, attn_r.reshape(_D)[:, None], 0.0)
    return ml.astype(jnp.float32), mr.astype(jnp.float32)


def kernel(x, edge_index_0, edge_index_1, W0, attn_l0, attn_r0, b0,
           W1, attn_l1, attn_r1, b1, Wp1, bp1, Wp2):
    src2 = jnp.stack([edge_index_0[0], edge_index_1[0]]).reshape(_NC * _R, _CHUNK)
    dst2 = jnp.stack([edge_index_0[1], edge_index_1[1]]).reshape(_NC * _R, _CHUNK)

    ml0, mr0 = _attn_mats(attn_l0, attn_r0)
    ml1, mr1 = _attn_mats(attn_l1, attn_r1)
    w = jnp.stack([W0, W1])
    ml = jnp.stack([ml0, ml1])
    mr = jnp.stack([mr0, mr1])
    bb = jnp.stack([b0, b1])

    feat, tl, tr, c = _pre(x, w, ml, mr)

    ee, dp = _p1(src2, dst2, tl.reshape(_NC * _N, 16),
                 tr.reshape(_NC * _N, 16), c.reshape(_NC, 16))
    op = _p2(src2, dst2, feat.reshape(_NC * _N, _D), ee)

    z0, z1, wsem = _ka(op, dp, bb, Wp1, bp1.reshape(1, _HID),
                       Wp2.reshape(1, _HID))
    return _kb(z0, z1, wsem)
